# baseline (speedup 1.0000x reference)
import functools

import jax
import jax.numpy as jnp
from jax import lax
from jax.experimental import pallas as pl
from jax.experimental.pallas import tpu as pltpu
from jax.experimental.pallas import tpu_sc as plsc

_NUM_EMB = 10000
_EMB_DIM = 128
_NUM_CLASSES = 3
_B = 16384
_L = 200

_LANES = 16
_NVEC = 13
_V_PAD = 10016
_NUM_WORKERS = 32
_ROWS_PER_W = _B // _NUM_WORKERS
_CHUNK = 64
_NCHUNK = _ROWS_PER_W // _CHUNK
_IWORDS = _CHUNK * _L


def _project_body(w_ref, emb_ref, out_ref):
    out_ref[...] = lax.dot_general(
        w_ref[...], emb_ref[...], (((1,), (1,)), ((), ())),
        preferred_element_type=jnp.float32)


def _project(fc_w_pad, emb_table):
    return pl.pallas_call(
        _project_body,
        out_shape=jax.ShapeDtypeStruct((8, _NUM_EMB), jnp.float32),
    )(fc_w_pad, emb_table)


_mesh = plsc.VectorSubcoreMesh(core_axis_name="c", subcore_axis_name="s")


@functools.partial(
    pl.kernel,
    out_type=jax.ShapeDtypeStruct((_B, _LANES), jnp.float32),
    mesh=_mesh,
    scratch_types=[
        pltpu.VMEM((_V_PAD,), jnp.float32),
        pltpu.VMEM((_V_PAD,), jnp.float32),
        pltpu.VMEM((_IWORDS + _LANES,), jnp.int32),
        pltpu.VMEM((_IWORDS + _LANES,), jnp.int32),
        pltpu.VMEM((2, _CHUNK, _LANES), jnp.float32),
        pltpu.VMEM((_LANES,), jnp.float32),
        pltpu.SemaphoreType.DMA,
        pltpu.SemaphoreType.DMA,
        pltpu.SemaphoreType.DMA,
    ],
    compiler_params=pltpu.CompilerParams(needs_layout_passes=False),
)
def _bag_kernel(p_hbm, textf_hbm, fcb_hbm, out_hbm,
                p01, p2, ibuf0, ibuf1, outbuf, biasbuf, sem0, sem1, osem):
    wid = lax.axis_index("s") * 2 + lax.axis_index("c")
    base = wid * _ROWS_PER_W

    safe = jnp.zeros((_LANES,), jnp.int32)
    ibuf0[pl.ds(_IWORDS, _LANES)] = safe
    ibuf1[pl.ds(_IWORDS, _LANES)] = safe

    pltpu.sync_copy(p_hbm.at[0], p01)
    pltpu.sync_copy(p_hbm.at[1], p2)
    pltpu.sync_copy(fcb_hbm, biasbuf)
    bvec = biasbuf[...]
    b0 = bvec[0]
    b1 = bvec[1]
    b2 = bvec[2]
    inv_l = jnp.float32(1.0 / _L)
    lane = lax.iota(jnp.int32, _LANES)
    m8 = lane < 8
    fzero = jnp.zeros((_LANES,), jnp.float32)
    himask = jnp.int32(-65536)

    ibufs = (ibuf0, ibuf1)
    sems = (sem0, sem1)

    def start_idx_copy(ci, slot):
        return pltpu.async_copy(
            textf_hbm.at[pl.ds((base + ci * _CHUNK) * _L, _IWORDS)],
            ibufs[slot].at[pl.ds(0, _IWORDS)], sems[slot])

    def bag_sums(ibuf, r):
        ro = pl.multiple_of(r * _L, 8)
        acc0 = fzero
        acc1 = fzero
        acc2 = fzero
        for j in range(_NVEC):
            idx = ibuf[pl.ds(ro + j * _LANES, _LANES)]
            g01 = plsc.bitcast(plsc.load_gather(p01, [idx]), jnp.int32)
            c0 = plsc.bitcast(g01 & himask, jnp.float32)
            c1 = plsc.bitcast(g01 << 16, jnp.float32)
            c2 = plsc.load_gather(p2, [idx])
            if j == _NVEC - 1:
                c0 = jnp.where(m8, c0, fzero)
                c1 = jnp.where(m8, c1, fzero)
                c2 = jnp.where(m8, c2, fzero)
            acc0 = acc0 + c0
            acc1 = acc1 + c1
            acc2 = acc2 + c2
        s0 = jnp.sum(acc0) * inv_l + b0
        s1 = jnp.sum(acc1) * inv_l + b1
        s2 = jnp.sum(acc2) * inv_l + b2
        return jnp.where(lane == 0, s0, jnp.where(lane == 1, s1, s2))

    def process_chunk(ci, slot):
        ibuf = ibufs[slot]

        def row_body(ri, carry2):
            r = ri * 2
            outbuf[slot, r, pl.ds(0, _LANES)] = bag_sums(ibuf, r)
            outbuf[slot, r + 1, pl.ds(0, _LANES)] = bag_sums(ibuf, r + 1)
            return carry2

        lax.fori_loop(0, _CHUNK // 2, row_body, 0)
        return pltpu.async_copy(
            outbuf.at[slot],
            out_hbm.at[pl.ds(base + ci * _CHUNK, _CHUNK)], osem)

    copies = [None, None]
    out_copies = [None, None]
    copies[0] = start_idx_copy(0, 0)
    for ci in range(_NCHUNK):
        slot = ci % 2
        if ci + 1 < _NCHUNK:
            copies[1 - slot] = start_idx_copy(ci + 1, 1 - slot)
        copies[slot].wait()
        if out_copies[slot] is not None:
            out_copies[slot].wait()
        out_copies[slot] = process_chunk(ci, slot)
    out_copies[0].wait()
    out_copies[1].wait()


def _pack_tables(p):
    u0 = lax.bitcast_convert_type(p[0].astype(jnp.bfloat16), jnp.uint16)
    u1 = lax.bitcast_convert_type(p[1].astype(jnp.bfloat16), jnp.uint16)
    w01 = (u0.astype(jnp.uint32) << 16) | u1.astype(jnp.uint32)
    p01 = lax.bitcast_convert_type(w01, jnp.float32)
    return jnp.stack([p01, p[2]])


def kernel(text, emb_table, fc_w, fc_b):
    text_flat = text.astype(jnp.int32).reshape(-1)
    fc_w_pad = jnp.pad(fc_w, ((0, 8 - _NUM_CLASSES), (0, 0)))
    fcb_pad = jnp.pad(fc_b, (0, _LANES - _NUM_CLASSES))
    p = _project(fc_w_pad, emb_table)
    p_pad = jnp.pad(p, ((0, 0), (0, _V_PAD - _NUM_EMB)))
    p_packed = _pack_tables(p_pad)
    out16 = _bag_kernel(p_packed, text_flat, fcb_pad)
    return out16[:, :_NUM_CLASSES]


# speedup vs baseline: 1.1934x; 1.1934x over previous
import functools

import jax
import jax.numpy as jnp
from jax import lax
from jax.experimental import pallas as pl
from jax.experimental.pallas import tpu as pltpu
from jax.experimental.pallas import tpu_sc as plsc

_NUM_EMB = 10000
_EMB_DIM = 128
_NUM_CLASSES = 3
_B = 16384
_L = 200

_LANES = 16
_NFULL = _L // _LANES
_V_PAD = 10016
_NUM_WORKERS = 32
_ROWS_PER_W = _B // _NUM_WORKERS
_CHUNK = 64
_NCHUNK = _ROWS_PER_W // _CHUNK


def _project_body(w_ref, emb_ref, out_ref):
    out_ref[...] = lax.dot_general(
        w_ref[...], emb_ref[...], (((1,), (1,)), ((), ())),
        preferred_element_type=jnp.float32)


def _project(fc_w_pad, emb_table):
    return pl.pallas_call(
        _project_body,
        out_shape=jax.ShapeDtypeStruct((8, _NUM_EMB), jnp.float32),
    )(fc_w_pad, emb_table)


_mesh = plsc.VectorSubcoreMesh(core_axis_name="c", subcore_axis_name="s")


@functools.partial(
    pl.kernel,
    out_type=jax.ShapeDtypeStruct((_B, _LANES), jnp.float32),
    mesh=_mesh,
    scratch_types=[
        pltpu.VMEM((_V_PAD,), jnp.float32),
        pltpu.VMEM((_V_PAD,), jnp.float32),
        pltpu.VMEM((2, _CHUNK, _L), jnp.int32),
        pltpu.VMEM((2, _CHUNK, _LANES), jnp.float32),
        pltpu.VMEM((_LANES,), jnp.float32),
        pltpu.SemaphoreType.DMA,
        pltpu.SemaphoreType.DMA,
        pltpu.SemaphoreType.DMA,
    ],
    compiler_params=pltpu.CompilerParams(needs_layout_passes=False),
)
def _bag_kernel(p_hbm, text_hbm, fcb_hbm, out_hbm,
                p01, p2, idxbuf, outbuf, biasbuf, sem0, sem1, osem):
    wid = lax.axis_index("s") * 2 + lax.axis_index("c")
    base = wid * _ROWS_PER_W

    pltpu.sync_copy(p_hbm.at[0], p01)
    pltpu.sync_copy(p_hbm.at[1], p2)
    pltpu.sync_copy(fcb_hbm, biasbuf)
    bvec = biasbuf[...]
    b0 = bvec[0]
    b1 = bvec[1]
    b2 = bvec[2]
    inv_l = jnp.float32(1.0 / _L)
    lane = lax.iota(jnp.int32, _LANES)
    mhi = lane >= 8
    fzero = jnp.zeros((_LANES,), jnp.float32)
    himask = jnp.int32(-65536)

    sems = (sem0, sem1)

    def start_idx_copy(ci, slot):
        return pltpu.async_copy(
            text_hbm.at[pl.ds(base + ci * _CHUNK, _CHUNK)],
            idxbuf.at[slot], sems[slot])

    def bag_sums(slot, r):
        acc0 = fzero
        acc1 = fzero
        acc2 = fzero
        for j in range(_NFULL + 1):
            off = j * _LANES if j < _NFULL else _L - _LANES
            idx = idxbuf[slot, r, pl.ds(off, _LANES)]
            g01 = plsc.bitcast(plsc.load_gather(p01, [idx]), jnp.int32)
            c0 = plsc.bitcast(g01 & himask, jnp.float32)
            c1 = plsc.bitcast(g01 << 16, jnp.float32)
            c2 = plsc.load_gather(p2, [idx])
            if j == _NFULL:
                c0 = jnp.where(mhi, c0, fzero)
                c1 = jnp.where(mhi, c1, fzero)
                c2 = jnp.where(mhi, c2, fzero)
            acc0 = acc0 + c0
            acc1 = acc1 + c1
            acc2 = acc2 + c2
        s0 = jnp.sum(acc0) * inv_l + b0
        s1 = jnp.sum(acc1) * inv_l + b1
        s2 = jnp.sum(acc2) * inv_l + b2
        return jnp.where(lane == 0, s0, jnp.where(lane == 1, s1, s2))

    def process_chunk(ci, slot):
        def row_body(ri, carry2):
            r = ri * 2
            outbuf[slot, r, pl.ds(0, _LANES)] = bag_sums(slot, r)
            outbuf[slot, r + 1, pl.ds(0, _LANES)] = bag_sums(slot, r + 1)
            return carry2

        lax.fori_loop(0, _CHUNK // 2, row_body, 0)
        return pltpu.async_copy(
            outbuf.at[slot],
            out_hbm.at[pl.ds(base + ci * _CHUNK, _CHUNK)], osem)

    copies = [None, None]
    out_copies = [None, None]
    copies[0] = start_idx_copy(0, 0)
    for ci in range(_NCHUNK):
        slot = ci % 2
        if ci + 1 < _NCHUNK:
            copies[1 - slot] = start_idx_copy(ci + 1, 1 - slot)
        copies[slot].wait()
        if out_copies[slot] is not None:
            out_copies[slot].wait()
        out_copies[slot] = process_chunk(ci, slot)
    out_copies[0].wait()
    out_copies[1].wait()


def _pack_tables(p):
    u0 = lax.bitcast_convert_type(p[0].astype(jnp.bfloat16), jnp.uint16)
    u1 = lax.bitcast_convert_type(p[1].astype(jnp.bfloat16), jnp.uint16)
    w01 = (u0.astype(jnp.uint32) << 16) | u1.astype(jnp.uint32)
    p01 = lax.bitcast_convert_type(w01, jnp.float32)
    return jnp.stack([p01, p[2]])


def kernel(text, emb_table, fc_w, fc_b):
    text_i32 = text.astype(jnp.int32)
    fc_w_pad = jnp.pad(fc_w, ((0, 8 - _NUM_CLASSES), (0, 0)))
    fcb_pad = jnp.pad(fc_b, (0, _LANES - _NUM_CLASSES))
    p = _project(fc_w_pad, emb_table)
    p_pad = jnp.pad(p, ((0, 0), (0, _V_PAD - _NUM_EMB)))
    p_packed = _pack_tables(p_pad)
    out16 = _bag_kernel(p_packed, text_i32, fcb_pad)
    return out16[:, :_NUM_CLASSES]
